# baseline (speedup 1.0000x reference)
import functools

import jax
import jax.numpy as jnp
from jax.experimental import pallas as pl
from jax.experimental.pallas import tpu as pltpu

_BM = 256


def _fused_kernel(x_ref, w_ref, b_ref, a_ref, o_ref, y_ref):
    @pl.when(pl.program_id(0) == 0)
    def _compute_y():
        y_ref[...] = jnp.dot(
            x_ref[...], w_ref[...], preferred_element_type=jnp.float32
        ).astype(jnp.bfloat16)

    acc = jnp.dot(a_ref[...].astype(jnp.bfloat16), y_ref[...],
                  preferred_element_type=jnp.float32)
    o_ref[...] = acc + b_ref[...]


@jax.jit
def kernel(X, A, W, b):
    n, d_in = X.shape
    d_out = W.shape[1]

    b2 = b.reshape(1, d_out)
    grid = (n // _BM,)
    out = pl.pallas_call(
        _fused_kernel,
        grid=grid,
        in_specs=[
            pl.BlockSpec((n, d_in), lambda i: (0, 0)),
            pl.BlockSpec((d_in, d_out), lambda i: (0, 0)),
            pl.BlockSpec((1, d_out), lambda i: (0, 0)),
            pl.BlockSpec((_BM, n), lambda i: (i, 0)),
        ],
        out_specs=pl.BlockSpec((_BM, d_out), lambda i: (i, 0)),
        out_shape=jax.ShapeDtypeStruct((n, d_out), jnp.float32),
        scratch_shapes=[pltpu.VMEM((n, d_out), jnp.bfloat16)],
        compiler_params=pltpu.CompilerParams(
            dimension_semantics=("arbitrary",),
        ),
    )(X, W, b2, A)
    return out


# speedup vs baseline: 1.0085x; 1.0085x over previous
import functools

import jax
import jax.numpy as jnp
from jax.experimental import pallas as pl
from jax.experimental.pallas import tpu as pltpu

_BM = 128
_NSPLIT = 4


def _fused_kernel(x_ref, w_ref, b_ref, *rest):
    a_refs = rest[:_NSPLIT]
    o_ref = rest[_NSPLIT]
    y_ref = rest[_NSPLIT + 1]

    @pl.when(pl.program_id(0) == 0)
    def _compute_y():
        y_ref[...] = jnp.dot(
            x_ref[...], w_ref[...], preferred_element_type=jnp.float32
        ).astype(jnp.bfloat16)

    h = a_refs[0].shape[1]
    acc = b_ref[...].astype(jnp.float32)
    for j, a_ref in enumerate(a_refs):
        acc += jnp.dot(a_ref[...].astype(jnp.bfloat16),
                       y_ref[j * h:(j + 1) * h, :],
                       preferred_element_type=jnp.float32)
    o_ref[...] = acc


@jax.jit
def kernel(X, A, W, b):
    n, d_in = X.shape
    d_out = W.shape[1]

    b2 = b.reshape(1, d_out)
    h = n // _NSPLIT
    grid = (n // _BM,)

    def _a_spec(j):
        return pl.BlockSpec((_BM, h), lambda i, j=j: (i, j))

    out = pl.pallas_call(
        _fused_kernel,
        grid=grid,
        in_specs=[
            pl.BlockSpec((n, d_in), lambda i: (0, 0)),
            pl.BlockSpec((d_in, d_out), lambda i: (0, 0)),
            pl.BlockSpec((1, d_out), lambda i: (0, 0)),
        ] + [_a_spec(j) for j in range(_NSPLIT)],
        out_specs=pl.BlockSpec((_BM, d_out), lambda i: (i, 0)),
        out_shape=jax.ShapeDtypeStruct((n, d_out), jnp.float32),
        scratch_shapes=[pltpu.VMEM((n, d_out), jnp.bfloat16)],
        compiler_params=pltpu.CompilerParams(
            dimension_semantics=("arbitrary",),
        ),
    )(X, W, b2, *([A] * _NSPLIT))
    return out
